# Initial kernel scaffold; baseline (speedup 1.0000x reference)
import jax
import jax.numpy as jnp
from jax.experimental import pallas as pl


def kernel(t, alpha, alpha_bar):
    raise NotImplementedError("write your pallas kernel here")


# speedup vs baseline: 4.5982x; 4.5982x over previous
import jax
import jax.numpy as jnp
from jax import lax
from jax.experimental import pallas as pl
from jax.experimental.pallas import tpu as pltpu
from jax.experimental.pallas import tpu_sc as plsc

_NC = 2
_NS = 16
_L = 16
_NW = _NC * _NS
_B = 16384
_BPW = _B // _NW
_TABLE = 1001


def _gather_body(tab_hbm, idx_hbm, out_hbm, tab_v, idx_v, out_v):
    wid = lax.axis_index("s") * _NC + lax.axis_index("c")
    base = wid * _BPW
    pltpu.sync_copy(tab_hbm, tab_v)
    pltpu.sync_copy(idx_hbm.at[pl.ds(base, _BPW)], idx_v)
    for i in range(_BPW // _L):
        idx = idx_v[pl.ds(i * _L, _L)]
        out_v[pl.ds(i * _L, _L)] = plsc.load_gather(tab_v, [idx])
    pltpu.sync_copy(out_v, out_hbm.at[pl.ds(base, _BPW)])


def kernel(t, alpha, alpha_bar):
    del alpha
    mesh = plsc.VectorSubcoreMesh(core_axis_name="c", subcore_axis_name="s")
    f = pl.kernel(
        _gather_body,
        out_type=jax.ShapeDtypeStruct((_B,), jnp.float32),
        mesh=mesh,
        scratch_types=[
            pltpu.VMEM((_TABLE,), jnp.float32),
            pltpu.VMEM((_BPW,), jnp.int32),
            pltpu.VMEM((_BPW,), jnp.float32),
        ],
        compiler_params=pltpu.CompilerParams(needs_layout_passes=False),
    )
    return f(alpha_bar, t)


# speedup vs baseline: 4.6741x; 1.0165x over previous
import jax
import jax.numpy as jnp
from jax import lax
from jax.experimental import pallas as pl
from jax.experimental.pallas import tpu as pltpu
from jax.experimental.pallas import tpu_sc as plsc

_NC = 2
_NS = 16
_L = 16
_NW = _NC * _NS
_B = 16384
_BPW = _B // _NW
_TABLE = 1001


def _gather_body(tab_hbm, idx_hbm, out_hbm, tab_v, idx_v, out_v, sem_t, sem_i):
    wid = lax.axis_index("s") * _NC + lax.axis_index("c")
    base = wid * _BPW
    ct = pltpu.async_copy(tab_hbm, tab_v, sem_t)
    ci = pltpu.async_copy(idx_hbm.at[pl.ds(base, _BPW)], idx_v, sem_i)
    ct.wait()
    ci.wait()
    for i in range(_BPW // _L):
        idx = idx_v[pl.ds(i * _L, _L)]
        out_v[pl.ds(i * _L, _L)] = plsc.load_gather(tab_v, [idx])
    pltpu.sync_copy(out_v, out_hbm.at[pl.ds(base, _BPW)])


def kernel(t, alpha, alpha_bar):
    del alpha
    mesh = plsc.VectorSubcoreMesh(core_axis_name="c", subcore_axis_name="s")
    f = pl.kernel(
        _gather_body,
        out_type=jax.ShapeDtypeStruct((_B,), jnp.float32),
        mesh=mesh,
        scratch_types=[
            pltpu.VMEM((_TABLE,), jnp.float32),
            pltpu.VMEM((_BPW,), jnp.int32),
            pltpu.VMEM((_BPW,), jnp.float32),
            pltpu.SemaphoreType.DMA,
            pltpu.SemaphoreType.DMA,
        ],
        compiler_params=pltpu.CompilerParams(needs_layout_passes=False),
    )
    return f(alpha_bar, t)


# speedup vs baseline: 4.6764x; 1.0005x over previous
import jax
import jax.numpy as jnp
from jax import lax
from jax.experimental import pallas as pl
from jax.experimental.pallas import tpu as pltpu
from jax.experimental.pallas import tpu_sc as plsc

_NC = 2
_NS = 16
_L = 16
_NW = _NC * _NS
_B = 16384
_BPW = _B // _NW
_TABLE = 1001


def _gather_body(tab_hbm, idx_hbm, out_hbm, tab_v, idx_v, out_v, sem_t, sem_i):
    wid = lax.axis_index("s") * _NC + lax.axis_index("c")
    base = wid * _BPW
    ct = pltpu.async_copy(tab_hbm, tab_v, sem_t)
    ci = pltpu.async_copy(idx_hbm.at[pl.ds(base, _BPW)], idx_v, sem_i)
    ct.wait()
    ci.wait()
    for i in range(_BPW // _L):
        idx = idx_v[pl.ds(i * _L, _L)]
        out_v[pl.ds(i * _L, _L)] = plsc.load_gather(tab_v, [idx])
    pltpu.sync_copy(out_v, out_hbm.at[pl.ds(base, _BPW)])


def kernel(t, alpha, alpha_bar):
    del alpha
    mesh = plsc.VectorSubcoreMesh(core_axis_name="c", subcore_axis_name="s")
    f = pl.kernel(
        _gather_body,
        out_type=jax.ShapeDtypeStruct((_B,), jnp.float32),
        mesh=mesh,
        scratch_types=[
            pltpu.VMEM((_TABLE,), jnp.float32),
            pltpu.VMEM((_BPW,), jnp.int32),
            pltpu.VMEM((_BPW,), jnp.float32),
            pltpu.SemaphoreType.DMA,
            pltpu.SemaphoreType.DMA,
        ],
        compiler_params=pltpu.CompilerParams(
            needs_layout_passes=False,
            disable_bounds_checks=True,
            disable_semaphore_checks=True,
            skip_device_barrier=True,
        ),
    )
    return f(alpha_bar, t)


# speedup vs baseline: 4.7022x; 1.0055x over previous
import jax
import jax.numpy as jnp
from jax import lax
from jax.experimental import pallas as pl
from jax.experimental.pallas import tpu as pltpu
from jax.experimental.pallas import tpu_sc as plsc

_NC = 2
_NS = 16
_L = 16
_NW = _NC * _NS
_B = 16384
_BPW = _B // _NW
_TABLE = 1001


def _gather_body(tab_hbm, idx_hbm, out_hbm, tab_v, idx_v, out_v, sem_t, sem_i):
    wid = lax.axis_index("s") * _NC + lax.axis_index("c")
    base = wid * _BPW
    ct = pltpu.async_copy(tab_hbm, tab_v, sem_t)
    ci = pltpu.async_copy(idx_hbm.at[pl.ds(base, _BPW)], idx_v, sem_i)
    ct.wait()
    ci.wait()
    def step(i, carry):
        off = i * _L
        idx = idx_v[pl.ds(off, _L)]
        out_v[pl.ds(off, _L)] = plsc.load_gather(tab_v, [idx])
        return carry

    lax.fori_loop(0, _BPW // _L, step, 0, unroll=4)
    pltpu.sync_copy(out_v, out_hbm.at[pl.ds(base, _BPW)])


def kernel(t, alpha, alpha_bar):
    del alpha
    mesh = plsc.VectorSubcoreMesh(core_axis_name="c", subcore_axis_name="s")
    f = pl.kernel(
        _gather_body,
        out_type=jax.ShapeDtypeStruct((_B,), jnp.float32),
        mesh=mesh,
        scratch_types=[
            pltpu.VMEM((_TABLE,), jnp.float32),
            pltpu.VMEM((_BPW,), jnp.int32),
            pltpu.VMEM((_BPW,), jnp.float32),
            pltpu.SemaphoreType.DMA,
            pltpu.SemaphoreType.DMA,
        ],
        compiler_params=pltpu.CompilerParams(
            needs_layout_passes=False,
            disable_bounds_checks=True,
            disable_semaphore_checks=True,
            skip_device_barrier=True,
        ),
    )
    return f(alpha_bar, t)
